# baseline (speedup 1.0000x reference)
import jax
import jax.numpy as jnp
from jax import lax
from jax.experimental import pallas as pl
from jax.experimental.pallas import tpu as pltpu
from jax.experimental.pallas import tpu_sc as plsc

N, D_IN, D_HID, N_CLUSTERS, D_AUX = 1024, 1024, 512, 1024, 64
C_SC = 128
C_TC = N_CLUSTERS - C_SC
BM = 128
BNE = 128
BS = 128


def _mlp_body(x_ref, w1_ref, b1_ref, w2_ref, b2_ref, z_ref):
    x = x_ref[...].astype(jnp.bfloat16)
    w1 = w1_ref[...].astype(jnp.bfloat16)
    h = jnp.maximum(
        jnp.dot(x, w1, preferred_element_type=jnp.float32)
        + b1_ref[...], 0.0)
    z_ref[...] = jnp.dot(h.astype(jnp.bfloat16),
                         w2_ref[...].astype(jnp.bfloat16),
                         preferred_element_type=jnp.float32) + b2_ref[...]


def _einsum_body(z_ref, cc_ref, out_ref):
    zt = z_ref[...][:, :, None]
    out_ref[...] = jnp.sum(cc_ref[...] * zt, axis=1)


def _lsm_body(*refs):
    ins, out_ref = refs[:-1], refs[-1]
    l = jnp.concatenate([r[...] for r in ins], axis=1)
    lt = l.T
    m = jnp.max(lt, axis=0)
    e = jnp.exp(lt - m[None, :])
    s = jnp.sum(e, axis=0)
    r = lt - (m + jnp.log(s))[None, :]
    out_ref[...] = r.T


def _mlp(inputs, W1, b1, W2, b2):
    return pl.pallas_call(
        _mlp_body,
        grid=(N // BM,),
        in_specs=[
            pl.BlockSpec((BM, D_IN), lambda i: (i, 0)),
            pl.BlockSpec((D_IN, D_HID), lambda i: (0, 0)),
            pl.BlockSpec((1, D_HID), lambda i: (0, 0)),
            pl.BlockSpec((D_HID, D_AUX), lambda i: (0, 0)),
            pl.BlockSpec((1, D_AUX), lambda i: (0, 0)),
        ],
        out_specs=pl.BlockSpec((BM, D_AUX), lambda i: (i, 0)),
        out_shape=jax.ShapeDtypeStruct((N, D_AUX), jnp.float32),
    )(inputs, W1, b1.reshape(1, D_HID), W2, b2.reshape(1, D_AUX))


BCE = 128


def _sc_body(cc_ref, z_ref, out_ref, buf0, buf1, zs0, zs1, ov0, ov1,
             csem0, csem1, zsem0, zsem1, osem0, osem1):
    nc = 2
    wid = jax.lax.axis_index("s") * nc + jax.lax.axis_index("c")
    base = wid * (N // 32)
    bufs = (buf0, buf1)
    zss = (zs0, zs1)
    ovs = (ov0, ov1)
    csems = (csem0, csem1)
    zsems = (zsem0, zsem1)
    osems = (osem0, osem1)
    nsamp = N // 32

    def cc_copy(n, b):
        return pltpu.make_async_copy(
            cc_ref.at[n, :, pl.ds(C_TC, C_SC)], bufs[b], csems[b])

    def z_copy(n, b):
        return pltpu.make_async_copy(z_ref.at[n], zss[b], zsems[b])

    for b in range(2):
        cc_copy(base + b, b).start()
        z_copy(base + b, b).start()

    def sample(i, b):
        n = base + i
        buf, zs, ov = bufs[b], zss[b], ovs[b]

        @pl.when(i >= 2)
        def _():
            pltpu.make_async_copy(ov, out_ref.at[n], osems[b]).wait()

        cc_copy(n, b).wait()
        z_copy(n, b).wait()
        lane_idx = [jnp.full((16, 1), l, jnp.int32) for l in range(16)]
        gdn = jax.lax.GatherDimensionNumbers(
            offset_dims=(), collapsed_slice_dims=(0,), start_index_map=(0,))

        def group(g, _):
            acc = jnp.zeros((16,), jnp.float32)
            sl = pl.ds(g * 16, 16)
            for dc in range(D_AUX // 16):
                zc = zs[pl.ds(dc * 16, 16)]
                for l in range(16):
                    zb = jax.lax.gather(
                        zc, lane_idx[l], gdn, slice_sizes=(1,),
                        mode=jax.lax.GatherScatterMode.PROMISE_IN_BOUNDS)
                    acc = acc + zb * buf[dc * 16 + l, sl]
            ov[sl] = acc
            return 0

        jax.lax.fori_loop(0, C_SC // 16, group, 0)
        nxt = base + jax.lax.rem(i + 2, nsamp)
        cc_copy(nxt, b).start()
        z_copy(nxt, b).start()
        pltpu.make_async_copy(ov, out_ref.at[n], osems[b]).start()
        return 0

    def pair(j, _):
        sample(j * 2, 0)
        sample(j * 2 + 1, 1)
        return 0

    jax.lax.fori_loop(0, nsamp // 2, pair, 0)
    for b in range(2):
        pltpu.make_async_copy(ovs[b], out_ref.at[base], osems[b]).wait()


def _einsum_sc(z, cc_t):
    mesh = plsc.VectorSubcoreMesh(core_axis_name="c", subcore_axis_name="s")
    f32 = jnp.float32
    return pl.kernel(
        _sc_body,
        out_type=jax.ShapeDtypeStruct((N, C_SC), f32),
        mesh=mesh,
        cost_estimate=pl.CostEstimate(
            flops=2 * N * C_SC * D_AUX,
            bytes_accessed=4 * N * C_SC * D_AUX,
            transcendentals=0),
        scratch_types=[
            pltpu.VMEM((D_AUX, C_SC), f32), pltpu.VMEM((D_AUX, C_SC), f32),
            pltpu.VMEM((D_AUX,), f32), pltpu.VMEM((D_AUX,), f32),
            pltpu.VMEM((C_SC,), f32), pltpu.VMEM((C_SC,), f32),
            pltpu.SemaphoreType.DMA, pltpu.SemaphoreType.DMA,
            pltpu.SemaphoreType.DMA, pltpu.SemaphoreType.DMA,
            pltpu.SemaphoreType.DMA, pltpu.SemaphoreType.DMA,
        ],
    )(cc_t, z)


def _einsum_tc(z, cc_t):
    return pl.pallas_call(
        _einsum_body,
        grid=(N // BNE, C_TC // BCE),
        in_specs=[
            pl.BlockSpec((BNE, D_AUX), lambda i, j: (i, 0)),
            pl.BlockSpec((BNE, D_AUX, BCE), lambda i, j: (i, 0, j)),
        ],
        out_specs=pl.BlockSpec((BNE, BCE), lambda i, j: (i, j)),
        out_shape=jax.ShapeDtypeStruct((N, C_TC), jnp.float32),
        cost_estimate=pl.CostEstimate(
            flops=2 * N * C_TC * D_AUX,
            bytes_accessed=4 * N * C_TC * D_AUX,
            transcendentals=0),
    )(z, cc_t)


def _log_softmax(parts):
    n_in = len(parts)
    widths = [p.shape[1] for p in parts]
    return pl.pallas_call(
        _lsm_body,
        grid=(N // BS,),
        in_specs=[pl.BlockSpec((BS, w), lambda i: (i, 0)) for w in widths],
        out_specs=pl.BlockSpec((BS, N_CLUSTERS), lambda i: (i, 0)),
        out_shape=jax.ShapeDtypeStruct((N, N_CLUSTERS), jnp.float32),
    )(*parts)


def kernel(inputs, cluster_centers, W1, b1, W2, b2):
    cc_t = jnp.swapaxes(cluster_centers, 1, 2)
    z = _mlp(inputs, W1, b1, W2, b2)
    parts = []
    if C_TC > 0:
        parts.append(_einsum_tc(z, cc_t))
    if C_SC > 0:
        parts.append(_einsum_sc(z, cc_t))
    return _log_softmax(parts)


# speedup vs baseline: 1.5386x; 1.5386x over previous
import jax
import jax.numpy as jnp
from jax import lax
from jax.experimental import pallas as pl
from jax.experimental.pallas import tpu as pltpu
from jax.experimental.pallas import tpu_sc as plsc

N, D_IN, D_HID, N_CLUSTERS, D_AUX = 1024, 1024, 512, 1024, 64
C_SC = 0
C_TC = N_CLUSTERS - C_SC
BM = 128
BNE = 128
BS = 128


def _mlp_body(x_ref, w1_ref, b1_ref, w2_ref, b2_ref, z_ref):
    x = x_ref[...].astype(jnp.bfloat16)
    w1 = w1_ref[...].astype(jnp.bfloat16)
    h = jnp.maximum(
        jnp.dot(x, w1, preferred_element_type=jnp.float32)
        + b1_ref[...], 0.0)
    z_ref[...] = jnp.dot(h.astype(jnp.bfloat16),
                         w2_ref[...].astype(jnp.bfloat16),
                         preferred_element_type=jnp.float32) + b2_ref[...]


def _einsum_body(z_ref, cc_ref, out_ref):
    zt = z_ref[...][:, :, None]
    out_ref[...] = jnp.sum(cc_ref[...] * zt, axis=1)


def _lsm_body(*refs):
    ins, out_ref = refs[:-1], refs[-1]
    l = jnp.concatenate([r[...] for r in ins], axis=1)
    lt = l.T
    m = jnp.max(lt, axis=0)
    e = jnp.exp(lt - m[None, :])
    s = jnp.sum(e, axis=0)
    r = lt - (m + jnp.log(s))[None, :]
    out_ref[...] = r.T


def _mlp(inputs, W1, b1, W2, b2):
    return pl.pallas_call(
        _mlp_body,
        grid=(N // BM,),
        in_specs=[
            pl.BlockSpec((BM, D_IN), lambda i: (i, 0)),
            pl.BlockSpec((D_IN, D_HID), lambda i: (0, 0)),
            pl.BlockSpec((1, D_HID), lambda i: (0, 0)),
            pl.BlockSpec((D_HID, D_AUX), lambda i: (0, 0)),
            pl.BlockSpec((1, D_AUX), lambda i: (0, 0)),
        ],
        out_specs=pl.BlockSpec((BM, D_AUX), lambda i: (i, 0)),
        out_shape=jax.ShapeDtypeStruct((N, D_AUX), jnp.float32),
    )(inputs, W1, b1.reshape(1, D_HID), W2, b2.reshape(1, D_AUX))


BCE = 128


def _sc_body(cc_ref, z_ref, out_ref, buf0, buf1, zs0, zs1, ov0, ov1,
             csem0, csem1, zsem0, zsem1, osem0, osem1):
    nc = 2
    wid = jax.lax.axis_index("s") * nc + jax.lax.axis_index("c")
    base = wid * (N // 32)
    bufs = (buf0, buf1)
    zss = (zs0, zs1)
    ovs = (ov0, ov1)
    csems = (csem0, csem1)
    zsems = (zsem0, zsem1)
    osems = (osem0, osem1)
    nsamp = N // 32

    def cc_copy(n, b):
        return pltpu.make_async_copy(
            cc_ref.at[n, :, pl.ds(C_TC, C_SC)], bufs[b], csems[b])

    def z_copy(n, b):
        return pltpu.make_async_copy(z_ref.at[n], zss[b], zsems[b])

    for b in range(2):
        cc_copy(base + b, b).start()
        z_copy(base + b, b).start()

    def sample(i, b):
        n = base + i
        buf, zs, ov = bufs[b], zss[b], ovs[b]

        @pl.when(i >= 2)
        def _():
            pltpu.make_async_copy(ov, out_ref.at[n], osems[b]).wait()

        cc_copy(n, b).wait()
        z_copy(n, b).wait()
        lane_idx = [jnp.full((16, 1), l, jnp.int32) for l in range(16)]
        gdn = jax.lax.GatherDimensionNumbers(
            offset_dims=(), collapsed_slice_dims=(0,), start_index_map=(0,))

        def group(g, _):
            acc = jnp.zeros((16,), jnp.float32)
            sl = pl.ds(g * 16, 16)
            for dc in range(D_AUX // 16):
                zc = zs[pl.ds(dc * 16, 16)]
                for l in range(16):
                    zb = jax.lax.gather(
                        zc, lane_idx[l], gdn, slice_sizes=(1,),
                        mode=jax.lax.GatherScatterMode.PROMISE_IN_BOUNDS)
                    acc = acc + zb * buf[dc * 16 + l, sl]
            ov[sl] = acc
            return 0

        jax.lax.fori_loop(0, C_SC // 16, group, 0)
        nxt = base + jax.lax.rem(i + 2, nsamp)
        cc_copy(nxt, b).start()
        z_copy(nxt, b).start()
        pltpu.make_async_copy(ov, out_ref.at[n], osems[b]).start()
        return 0

    def pair(j, _):
        sample(j * 2, 0)
        sample(j * 2 + 1, 1)
        return 0

    jax.lax.fori_loop(0, nsamp // 2, pair, 0)
    for b in range(2):
        pltpu.make_async_copy(ovs[b], out_ref.at[base], osems[b]).wait()


def _einsum_sc(z, cc_t):
    mesh = plsc.VectorSubcoreMesh(core_axis_name="c", subcore_axis_name="s")
    f32 = jnp.float32
    return pl.kernel(
        _sc_body,
        out_type=jax.ShapeDtypeStruct((N, C_SC), f32),
        mesh=mesh,
        cost_estimate=pl.CostEstimate(
            flops=2 * N * C_SC * D_AUX,
            bytes_accessed=4 * N * C_SC * D_AUX,
            transcendentals=0),
        scratch_types=[
            pltpu.VMEM((D_AUX, C_SC), f32), pltpu.VMEM((D_AUX, C_SC), f32),
            pltpu.VMEM((D_AUX,), f32), pltpu.VMEM((D_AUX,), f32),
            pltpu.VMEM((C_SC,), f32), pltpu.VMEM((C_SC,), f32),
            pltpu.SemaphoreType.DMA, pltpu.SemaphoreType.DMA,
            pltpu.SemaphoreType.DMA, pltpu.SemaphoreType.DMA,
            pltpu.SemaphoreType.DMA, pltpu.SemaphoreType.DMA,
        ],
    )(cc_t, z)


def _einsum_tc(z, cc_t):
    return pl.pallas_call(
        _einsum_body,
        grid=(N // BNE, C_TC // BCE),
        in_specs=[
            pl.BlockSpec((BNE, D_AUX), lambda i, j: (i, 0)),
            pl.BlockSpec((BNE, D_AUX, BCE), lambda i, j: (i, 0, j)),
        ],
        out_specs=pl.BlockSpec((BNE, BCE), lambda i, j: (i, j)),
        out_shape=jax.ShapeDtypeStruct((N, C_TC), jnp.float32),
        cost_estimate=pl.CostEstimate(
            flops=2 * N * C_TC * D_AUX,
            bytes_accessed=4 * N * C_TC * D_AUX,
            transcendentals=0),
    )(z, cc_t)


def _log_softmax(parts):
    n_in = len(parts)
    widths = [p.shape[1] for p in parts]
    return pl.pallas_call(
        _lsm_body,
        grid=(N // BS,),
        in_specs=[pl.BlockSpec((BS, w), lambda i: (i, 0)) for w in widths],
        out_specs=pl.BlockSpec((BS, N_CLUSTERS), lambda i: (i, 0)),
        out_shape=jax.ShapeDtypeStruct((N, N_CLUSTERS), jnp.float32),
    )(*parts)


BNF = 32


def _fused_full_body(x_ref, cc_ref, w1_ref, b1_ref, w2_ref, b2_ref, o_ref):
    x = x_ref[...].astype(jnp.bfloat16)
    w1 = w1_ref[...].astype(jnp.bfloat16)
    h = jnp.maximum(
        jnp.dot(x, w1, preferred_element_type=jnp.float32)
        + b1_ref[...], 0.0)
    z = jnp.dot(h.astype(jnp.bfloat16), w2_ref[...].astype(jnp.bfloat16),
                preferred_element_type=jnp.float32) + b2_ref[...]
    zt = z[:, :, None]
    logits = jnp.sum(cc_ref[...] * zt, axis=1)
    lt = logits.T
    m = jnp.max(lt, axis=0)
    e = jnp.exp(lt - m[None, :])
    s = jnp.sum(e, axis=0)
    o_ref[...] = (lt - (m + jnp.log(s))[None, :]).T


def _fused_full(inputs, cc_t, W1, b1, W2, b2):
    return pl.pallas_call(
        _fused_full_body,
        grid=(N // BNF,),
        in_specs=[
            pl.BlockSpec((BNF, D_IN), lambda i: (i, 0)),
            pl.BlockSpec((BNF, D_AUX, N_CLUSTERS), lambda i: (i, 0, 0)),
            pl.BlockSpec((D_IN, D_HID), lambda i: (0, 0)),
            pl.BlockSpec((1, D_HID), lambda i: (0, 0)),
            pl.BlockSpec((D_HID, D_AUX), lambda i: (0, 0)),
            pl.BlockSpec((1, D_AUX), lambda i: (0, 0)),
        ],
        out_specs=pl.BlockSpec((BNF, N_CLUSTERS), lambda i: (i, 0)),
        out_shape=jax.ShapeDtypeStruct((N, N_CLUSTERS), jnp.float32),
    )(inputs, cc_t, W1, b1.reshape(1, D_HID), W2, b2.reshape(1, D_AUX))


def kernel(inputs, cluster_centers, W1, b1, W2, b2):
    if C_SC == 0:
        cc_t = jnp.swapaxes(cluster_centers, 1, 2)
        return _fused_full(inputs, cc_t, W1, b1, W2, b2)
    cc_t = jnp.swapaxes(cluster_centers, 1, 2)
    z = _mlp(inputs, W1, b1, W2, b2)
    parts = []
    if C_TC > 0:
        parts.append(_einsum_tc(z, cc_t))
    if C_SC > 0:
        parts.append(_einsum_sc(z, cc_t))
    return _log_softmax(parts)


# speedup vs baseline: 1.6185x; 1.0519x over previous
import jax
import jax.numpy as jnp
from jax import lax
from jax.experimental import pallas as pl
from jax.experimental.pallas import tpu as pltpu
from jax.experimental.pallas import tpu_sc as plsc

N, D_IN, D_HID, N_CLUSTERS, D_AUX = 1024, 1024, 512, 1024, 64
C_SC = 0
C_TC = N_CLUSTERS - C_SC
BM = 128
BNE = 128
BS = 128


def _mlp_body(x_ref, w1_ref, b1_ref, w2_ref, b2_ref, z_ref):
    x = x_ref[...].astype(jnp.bfloat16)
    w1 = w1_ref[...].astype(jnp.bfloat16)
    h = jnp.maximum(
        jnp.dot(x, w1, preferred_element_type=jnp.float32)
        + b1_ref[...], 0.0)
    z_ref[...] = jnp.dot(h.astype(jnp.bfloat16),
                         w2_ref[...].astype(jnp.bfloat16),
                         preferred_element_type=jnp.float32) + b2_ref[...]


def _einsum_body(z_ref, cc_ref, out_ref):
    zt = z_ref[...][:, :, None]
    out_ref[...] = jnp.sum(cc_ref[...] * zt, axis=1)


def _lsm_body(*refs):
    ins, out_ref = refs[:-1], refs[-1]
    l = jnp.concatenate([r[...] for r in ins], axis=1)
    lt = l.T
    m = jnp.max(lt, axis=0)
    e = jnp.exp(lt - m[None, :])
    s = jnp.sum(e, axis=0)
    r = lt - (m + jnp.log(s))[None, :]
    out_ref[...] = r.T


def _mlp(inputs, W1, b1, W2, b2):
    return pl.pallas_call(
        _mlp_body,
        grid=(N // BM,),
        in_specs=[
            pl.BlockSpec((BM, D_IN), lambda i: (i, 0)),
            pl.BlockSpec((D_IN, D_HID), lambda i: (0, 0)),
            pl.BlockSpec((1, D_HID), lambda i: (0, 0)),
            pl.BlockSpec((D_HID, D_AUX), lambda i: (0, 0)),
            pl.BlockSpec((1, D_AUX), lambda i: (0, 0)),
        ],
        out_specs=pl.BlockSpec((BM, D_AUX), lambda i: (i, 0)),
        out_shape=jax.ShapeDtypeStruct((N, D_AUX), jnp.float32),
    )(inputs, W1, b1.reshape(1, D_HID), W2, b2.reshape(1, D_AUX))


BCE = 128


def _sc_body(cc_ref, z_ref, out_ref, buf0, buf1, zs0, zs1, ov0, ov1,
             csem0, csem1, zsem0, zsem1, osem0, osem1):
    nc = 2
    wid = jax.lax.axis_index("s") * nc + jax.lax.axis_index("c")
    base = wid * (N // 32)
    bufs = (buf0, buf1)
    zss = (zs0, zs1)
    ovs = (ov0, ov1)
    csems = (csem0, csem1)
    zsems = (zsem0, zsem1)
    osems = (osem0, osem1)
    nsamp = N // 32

    def cc_copy(n, b):
        return pltpu.make_async_copy(
            cc_ref.at[n, :, pl.ds(C_TC, C_SC)], bufs[b], csems[b])

    def z_copy(n, b):
        return pltpu.make_async_copy(z_ref.at[n], zss[b], zsems[b])

    for b in range(2):
        cc_copy(base + b, b).start()
        z_copy(base + b, b).start()

    def sample(i, b):
        n = base + i
        buf, zs, ov = bufs[b], zss[b], ovs[b]

        @pl.when(i >= 2)
        def _():
            pltpu.make_async_copy(ov, out_ref.at[n], osems[b]).wait()

        cc_copy(n, b).wait()
        z_copy(n, b).wait()
        lane_idx = [jnp.full((16, 1), l, jnp.int32) for l in range(16)]
        gdn = jax.lax.GatherDimensionNumbers(
            offset_dims=(), collapsed_slice_dims=(0,), start_index_map=(0,))

        def group(g, _):
            acc = jnp.zeros((16,), jnp.float32)
            sl = pl.ds(g * 16, 16)
            for dc in range(D_AUX // 16):
                zc = zs[pl.ds(dc * 16, 16)]
                for l in range(16):
                    zb = jax.lax.gather(
                        zc, lane_idx[l], gdn, slice_sizes=(1,),
                        mode=jax.lax.GatherScatterMode.PROMISE_IN_BOUNDS)
                    acc = acc + zb * buf[dc * 16 + l, sl]
            ov[sl] = acc
            return 0

        jax.lax.fori_loop(0, C_SC // 16, group, 0)
        nxt = base + jax.lax.rem(i + 2, nsamp)
        cc_copy(nxt, b).start()
        z_copy(nxt, b).start()
        pltpu.make_async_copy(ov, out_ref.at[n], osems[b]).start()
        return 0

    def pair(j, _):
        sample(j * 2, 0)
        sample(j * 2 + 1, 1)
        return 0

    jax.lax.fori_loop(0, nsamp // 2, pair, 0)
    for b in range(2):
        pltpu.make_async_copy(ovs[b], out_ref.at[base], osems[b]).wait()


def _einsum_sc(z, cc_t):
    mesh = plsc.VectorSubcoreMesh(core_axis_name="c", subcore_axis_name="s")
    f32 = jnp.float32
    return pl.kernel(
        _sc_body,
        out_type=jax.ShapeDtypeStruct((N, C_SC), f32),
        mesh=mesh,
        cost_estimate=pl.CostEstimate(
            flops=2 * N * C_SC * D_AUX,
            bytes_accessed=4 * N * C_SC * D_AUX,
            transcendentals=0),
        scratch_types=[
            pltpu.VMEM((D_AUX, C_SC), f32), pltpu.VMEM((D_AUX, C_SC), f32),
            pltpu.VMEM((D_AUX,), f32), pltpu.VMEM((D_AUX,), f32),
            pltpu.VMEM((C_SC,), f32), pltpu.VMEM((C_SC,), f32),
            pltpu.SemaphoreType.DMA, pltpu.SemaphoreType.DMA,
            pltpu.SemaphoreType.DMA, pltpu.SemaphoreType.DMA,
            pltpu.SemaphoreType.DMA, pltpu.SemaphoreType.DMA,
        ],
    )(cc_t, z)


def _einsum_tc(z, cc_t):
    return pl.pallas_call(
        _einsum_body,
        grid=(N // BNE, C_TC // BCE),
        in_specs=[
            pl.BlockSpec((BNE, D_AUX), lambda i, j: (i, 0)),
            pl.BlockSpec((BNE, D_AUX, BCE), lambda i, j: (i, 0, j)),
        ],
        out_specs=pl.BlockSpec((BNE, BCE), lambda i, j: (i, j)),
        out_shape=jax.ShapeDtypeStruct((N, C_TC), jnp.float32),
        cost_estimate=pl.CostEstimate(
            flops=2 * N * C_TC * D_AUX,
            bytes_accessed=4 * N * C_TC * D_AUX,
            transcendentals=0),
    )(z, cc_t)


def _log_softmax(parts):
    n_in = len(parts)
    widths = [p.shape[1] for p in parts]
    return pl.pallas_call(
        _lsm_body,
        grid=(N // BS,),
        in_specs=[pl.BlockSpec((BS, w), lambda i: (i, 0)) for w in widths],
        out_specs=pl.BlockSpec((BS, N_CLUSTERS), lambda i: (i, 0)),
        out_shape=jax.ShapeDtypeStruct((N, N_CLUSTERS), jnp.float32),
    )(*parts)


BNF = 64


def _fused_full_body(x_ref, cc_ref, w1_ref, b1_ref, w2_ref, b2_ref, o_ref):
    x = x_ref[...].astype(jnp.bfloat16)
    w1 = w1_ref[...].astype(jnp.bfloat16)
    h = jnp.maximum(
        jnp.dot(x, w1, preferred_element_type=jnp.float32)
        + b1_ref[...], 0.0)
    z = jnp.dot(h.astype(jnp.bfloat16), w2_ref[...].astype(jnp.bfloat16),
                preferred_element_type=jnp.float32) + b2_ref[...]
    zt = z[:, :, None]
    logits = jnp.sum(cc_ref[...] * zt, axis=1)
    lt = logits.T
    m = jnp.max(lt, axis=0)
    e = jnp.exp(lt - m[None, :])
    s = jnp.sum(e, axis=0)
    o_ref[...] = (lt - (m + jnp.log(s))[None, :]).T


def _fused_full(inputs, cc_t, W1, b1, W2, b2):
    return pl.pallas_call(
        _fused_full_body,
        grid=(N // BNF,),
        in_specs=[
            pl.BlockSpec((BNF, D_IN), lambda i: (i, 0)),
            pl.BlockSpec((BNF, D_AUX, N_CLUSTERS), lambda i: (i, 0, 0)),
            pl.BlockSpec((D_IN, D_HID), lambda i: (0, 0)),
            pl.BlockSpec((1, D_HID), lambda i: (0, 0)),
            pl.BlockSpec((D_HID, D_AUX), lambda i: (0, 0)),
            pl.BlockSpec((1, D_AUX), lambda i: (0, 0)),
        ],
        out_specs=pl.BlockSpec((BNF, N_CLUSTERS), lambda i: (i, 0)),
        out_shape=jax.ShapeDtypeStruct((N, N_CLUSTERS), jnp.float32),
    )(inputs, cc_t, W1, b1.reshape(1, D_HID), W2, b2.reshape(1, D_AUX))


def kernel(inputs, cluster_centers, W1, b1, W2, b2):
    if C_SC == 0:
        cc_t = jnp.swapaxes(cluster_centers, 1, 2)
        return _fused_full(inputs, cc_t, W1, b1, W2, b2)
    cc_t = jnp.swapaxes(cluster_centers, 1, 2)
    z = _mlp(inputs, W1, b1, W2, b2)
    parts = []
    if C_TC > 0:
        parts.append(_einsum_tc(z, cc_t))
    if C_SC > 0:
        parts.append(_einsum_sc(z, cc_t))
    return _log_softmax(parts)
